# Initial kernel scaffold; baseline (speedup 1.0000x reference)
import functools

import jax
import jax.numpy as jnp
import numpy as np
from jax import lax
from jax.experimental import pallas as pl
from jax.experimental.pallas import tpu as pltpu
from jax.experimental.pallas import tpu_sc as plsc

N = 10000
E = 320000
D_IN = 128
D_EDGE = 16
H1 = 128
H2 = 64
BN_EPS = 1e-5

NC = 2
NS = 16
L = 16
NW = NC * NS
EPT = E // NW
CHW = 80
NCH = EPT // CHW

_SC_MESH = dict(core_axis_name="c", subcore_axis_name="s")


def _make_sc_edge(P):
    R = P + 16
    NV = P // 16
    rows_per_tile = N // NS

    @functools.partial(
        pl.kernel,
        out_type=jax.ShapeDtypeStruct((NC, N, R), jnp.float32),
        mesh=plsc.VectorSubcoreMesh(**_SC_MESH),
        scratch_types=[
            pltpu.VMEM((CHW,), jnp.int32),
            pltpu.VMEM((CHW,), jnp.int32),
            pltpu.VMEM((CHW, P), jnp.float32),
            pltpu.VMEM((CHW, P), jnp.float32),
            pltpu.VMEM((CHW, P), jnp.float32),
            pltpu.VMEM((CHW, R), jnp.float32),
            pltpu.VMEM((P,), jnp.float32),
            pltpu.VMEM_SHARED((N, R), jnp.float32),
            pltpu.SemaphoreType.DMA,
            pltpu.SemaphoreType.DMA,
            pltpu.SemaphoreType.DMA,
        ],
    )
    def sc_edge(xl_hbm, xr_hbm, ee_hbm, src_hbm, dst_hbm, att_hbm, out_hbm,
                src_v, dst_v, xlb, xrb, eeb, outb, attb, acc_sh,
                sem1, sem2, sem3):
        cid = lax.axis_index("c")
        sid = lax.axis_index("s")
        wid = cid * NS + sid

        def zrow(i, _):
            for j in range(R // 16):
                outb[i, pl.ds(16 * j, 16)] = jnp.zeros((L,), jnp.float32)
            return 0
        lax.fori_loop(0, CHW, zrow, 0)
        row0 = sid * rows_per_tile
        nfull = rows_per_tile // CHW
        for k in range(nfull):
            pltpu.sync_copy(outb, acc_sh.at[pl.ds(row0 + k * CHW, CHW)])
        rem = rows_per_tile - nfull * CHW
        if rem:
            pltpu.sync_copy(outb.at[pl.ds(0, rem)],
                            acc_sh.at[pl.ds(row0 + nfull * CHW, rem)])
        pltpu.sync_copy(att_hbm, attb)
        plsc.subcore_barrier()

        def chunk(ch, _):
            base = wid * EPT + ch * CHW
            pltpu.sync_copy(src_hbm.at[pl.ds(base, CHW)], src_v)
            pltpu.sync_copy(dst_hbm.at[pl.ds(base, CHW)], dst_v)
            cp1 = pltpu.async_copy(xl_hbm.at[src_v], xlb, sem1)
            cp2 = pltpu.async_copy(xr_hbm.at[dst_v], xrb, sem2)
            cp3 = pltpu.async_copy(ee_hbm.at[pl.ds(base, CHW)], eeb, sem3)
            cp1.wait()
            cp2.wait()
            cp3.wait()

            def edge(e, _):
                s = jnp.zeros((L,), jnp.float32)
                xls = []
                for j in range(NV):
                    xlv = xlb[e, pl.ds(16 * j, 16)]
                    u = xlv + xrb[e, pl.ds(16 * j, 16)] + eeb[e, pl.ds(16 * j, 16)]
                    m = jnp.maximum(u, 0.2 * u)
                    s = s + m * attb[pl.ds(16 * j, 16)]
                    xls.append(xlv)
                ex = jnp.exp(jnp.full((L,), jnp.sum(s), jnp.float32))
                for j in range(NV):
                    outb[e, pl.ds(16 * j, 16)] = xls[j] * ex
                outb[e, pl.ds(P, 16)] = ex
                return 0

            lax.fori_loop(0, CHW, edge, 0)
            pltpu.sync_copy(outb, acc_sh.at[dst_v], add=True)
            return 0

        lax.fori_loop(0, NCH, chunk, 0)
        plsc.subcore_barrier()
        pltpu.sync_copy(acc_sh.at[pl.ds(row0, rows_per_tile)],
                        out_hbm.at[cid, pl.ds(row0, rows_per_tile)])

    return sc_edge


_sc_edge_128 = _make_sc_edge(128)
_sc_edge_64 = _make_sc_edge(64)


_BN_ROWS = 1000
_BE_ROWS = 1000


def _ee4(edge_attr, we0, we1, we2, we3):
    douts = (H1, H2, H1, D_IN)

    def body(ea_ref, w0, w1, w2, w3, o0, o1, o2, o3):
        ea = ea_ref[...]
        o0[...] = jnp.dot(ea, w0[...], preferred_element_type=jnp.float32)
        o1[...] = jnp.dot(ea, w1[...], preferred_element_type=jnp.float32)
        o2[...] = jnp.dot(ea, w2[...], preferred_element_type=jnp.float32)
        o3[...] = jnp.dot(ea, w3[...], preferred_element_type=jnp.float32)

    return pl.pallas_call(
        body,
        grid=(E // _BE_ROWS,),
        in_specs=[
            pl.BlockSpec((_BE_ROWS, D_EDGE), lambda i: (i, 0)),
            pl.BlockSpec((D_EDGE, H1), lambda i: (0, 0)),
            pl.BlockSpec((D_EDGE, H2), lambda i: (0, 0)),
            pl.BlockSpec((D_EDGE, H1), lambda i: (0, 0)),
            pl.BlockSpec((D_EDGE, D_IN), lambda i: (0, 0)),
        ],
        out_specs=[pl.BlockSpec((_BE_ROWS, d), lambda i: (i, 0)) for d in douts],
        out_shape=[jax.ShapeDtypeStruct((E, d), jnp.float32) for d in douts],
    )(edge_attr, we0, we1, we2, we3)


def _mm2(x, wl, wr):
    din, dout = wl.shape

    def body(x_ref, wl_ref, wr_ref, xl_ref, xr_ref):
        xb = x_ref[...]
        xl_ref[...] = jnp.dot(xb, wl_ref[...], preferred_element_type=jnp.float32)
        xr_ref[...] = jnp.dot(xb, wr_ref[...], preferred_element_type=jnp.float32)

    return pl.pallas_call(
        body,
        grid=(N // _BN_ROWS,),
        in_specs=[
            pl.BlockSpec((_BN_ROWS, din), lambda i: (i, 0)),
            pl.BlockSpec((din, dout), lambda i: (0, 0)),
            pl.BlockSpec((din, dout), lambda i: (0, 0)),
        ],
        out_specs=[pl.BlockSpec((_BN_ROWS, dout), lambda i: (i, 0))] * 2,
        out_shape=[jax.ShapeDtypeStruct((N, dout), jnp.float32)] * 2,
    )(x, wl, wr)


def _stage(acc, b_prev, g, bb, wl, wr, act):
    P = b_prev.shape[1]
    R = P + 16
    dout = wl.shape[1]
    gain = float(1.0 / np.sqrt(1.0 + BN_EPS))

    def body(acc_ref, b_ref, g_ref, bb_ref, wl_ref, wr_ref,
             h_ref, xl_ref, xr_ref):
        a = acc_ref[0] + acc_ref[1]
        h = a[:, :P] / (a[:, P:P + 1] + 1e-16) + b_ref[...]
        if act:
            h = jnp.maximum(h, 0.0) * (g_ref[...] * gain) + bb_ref[...]
        h_ref[...] = h
        xl_ref[...] = jnp.dot(h, wl_ref[...], preferred_element_type=jnp.float32)
        xr_ref[...] = jnp.dot(h, wr_ref[...], preferred_element_type=jnp.float32)

    return pl.pallas_call(
        body,
        grid=(N // _BN_ROWS,),
        in_specs=[
            pl.BlockSpec((2, _BN_ROWS, R), lambda i: (0, i, 0)),
            pl.BlockSpec((1, P), lambda i: (0, 0)),
            pl.BlockSpec((1, P), lambda i: (0, 0)),
            pl.BlockSpec((1, P), lambda i: (0, 0)),
            pl.BlockSpec((P, dout), lambda i: (0, 0)),
            pl.BlockSpec((P, dout), lambda i: (0, 0)),
        ],
        out_specs=[
            pl.BlockSpec((_BN_ROWS, P), lambda i: (i, 0)),
            pl.BlockSpec((_BN_ROWS, dout), lambda i: (i, 0)),
            pl.BlockSpec((_BN_ROWS, dout), lambda i: (i, 0)),
        ],
        out_shape=[
            jax.ShapeDtypeStruct((N, P), jnp.float32),
            jax.ShapeDtypeStruct((N, dout), jnp.float32),
            jax.ShapeDtypeStruct((N, dout), jnp.float32),
        ],
    )(acc, b_prev, g, bb, wl, wr)


def _finalize(acc, b_prev):
    P = b_prev.shape[1]
    R = P + 16

    def body(acc_ref, b_ref, o_ref):
        a = acc_ref[0] + acc_ref[1]
        o_ref[...] = a[:, :P] / (a[:, P:P + 1] + 1e-16) + b_ref[...]

    return pl.pallas_call(
        body,
        grid=(N // _BN_ROWS,),
        in_specs=[
            pl.BlockSpec((2, _BN_ROWS, R), lambda i: (0, i, 0)),
            pl.BlockSpec((1, P), lambda i: (0, 0)),
        ],
        out_specs=pl.BlockSpec((_BN_ROWS, P), lambda i: (i, 0)),
        out_shape=jax.ShapeDtypeStruct((N, P), jnp.float32),
    )(acc, b_prev)


def kernel(x, edge_index, edge_attr, params):
    src = edge_index[0].astype(jnp.int32)
    dst = edge_index[1].astype(jnp.int32)
    p0, p1 = params['enc0'], params['enc1']
    p2, p3 = params['dec0'], params['dec1']
    row = lambda v: v.reshape(1, -1)

    ee0, ee1, ee2, ee3 = _ee4(edge_attr, p0['We'], p1['We'], p2['We'], p3['We'])
    xl0, xr0 = _mm2(x, p0['Wl'], p0['Wr'])

    acc0 = _sc_edge_128(xl0, xr0, ee0, src, dst, p0['att'])
    h1, xl1, xr1 = _stage(acc0, row(p0['b']),
                          row(params['bn_e0']['g']), row(params['bn_e0']['b']),
                          p1['Wl'], p1['Wr'], act=True)
    acc1 = _sc_edge_64(xl1, xr1, ee1, src, dst, p1['att'])
    z, xl2, xr2 = _stage(acc1, row(p1['b']),
                         row(p1['b']), row(p1['b']),
                         p2['Wl'], p2['Wr'], act=False)
    acc2 = _sc_edge_128(xl2, xr2, ee2, src, dst, p2['att'])
    h2, xl3, xr3 = _stage(acc2, row(p2['b']),
                          row(params['bn_d0']['g']), row(params['bn_d0']['b']),
                          p3['Wl'], p3['Wr'], act=True)
    acc3 = _sc_edge_128(xl3, xr3, ee3, src, dst, p3['att'])
    x_rec = _finalize(acc3, row(p3['b']))
    return z, x_rec


# speedup vs baseline: 7.0289x; 7.0289x over previous
import functools

import jax
import jax.numpy as jnp
import numpy as np
from jax import lax
from jax.experimental import pallas as pl
from jax.experimental.pallas import tpu as pltpu
from jax.experimental.pallas import tpu_sc as plsc

N = 10000
E = 320000
D_IN = 128
D_EDGE = 16
H1 = 128
H2 = 64
BN_EPS = 1e-5

NC = 2
NS = 16
L = 16
NW = NC * NS
EPT = E // NW
CHW = 80
NCH = EPT // CHW

_SC_MESH = dict(core_axis_name="c", subcore_axis_name="s")


def _make_sc_edge(P):
    NV = P // 16
    rows_per_tile = N // NS

    @functools.partial(
        pl.kernel,
        out_type=(jax.ShapeDtypeStruct((NC, N, P), jnp.float32),
                  jax.ShapeDtypeStruct((NC, N, 16), jnp.float32)),
        mesh=plsc.VectorSubcoreMesh(**_SC_MESH),
        compiler_params=pltpu.CompilerParams(use_tc_tiling_on_sc=False,
                                             needs_layout_passes=False),
        scratch_types=[
            pltpu.VMEM((CHW,), jnp.int32),
            pltpu.VMEM((CHW,), jnp.int32),
            pltpu.VMEM((CHW, P), jnp.float32),
            pltpu.VMEM((CHW, P), jnp.float32),
            pltpu.VMEM((CHW, P), jnp.float32),
            pltpu.VMEM((CHW, 16), jnp.float32),
            pltpu.VMEM((P,), jnp.float32),
            pltpu.VMEM_SHARED((N, P), jnp.float32),
            pltpu.VMEM_SHARED((N, 16), jnp.float32),
            pltpu.SemaphoreType.DMA,
            pltpu.SemaphoreType.DMA,
            pltpu.SemaphoreType.DMA,
        ],
    )
    def sc_edge(xl_hbm, xr_hbm, ee_hbm, src_hbm, dst_hbm, att_hbm,
                out_hbm, outd_hbm,
                src_v, dst_v, xlb, xrb, eeb, exb, attb, acc_sh, den_sh,
                sem1, sem2, sem3):
        cid = lax.axis_index("c")
        sid = lax.axis_index("s")
        wid = cid * NS + sid

        def zrow(i, _):
            for j in range(NV):
                xlb[i, pl.ds(16 * j, 16)] = jnp.zeros((L,), jnp.float32)
            exb[i, :] = jnp.zeros((L,), jnp.float32)
            return 0
        lax.fori_loop(0, CHW, zrow, 0)
        row0 = sid * rows_per_tile
        nfull = rows_per_tile // CHW
        for k in range(nfull):
            pltpu.sync_copy(xlb, acc_sh.at[pl.ds(row0 + k * CHW, CHW)])
            pltpu.sync_copy(exb, den_sh.at[pl.ds(row0 + k * CHW, CHW)])
        rem = rows_per_tile - nfull * CHW
        if rem:
            pltpu.sync_copy(xlb.at[pl.ds(0, rem)],
                            acc_sh.at[pl.ds(row0 + nfull * CHW, rem)])
            pltpu.sync_copy(exb.at[pl.ds(0, rem)],
                            den_sh.at[pl.ds(row0 + nfull * CHW, rem)])
        pltpu.sync_copy(att_hbm, attb)
        plsc.subcore_barrier()

        def chunk(ch, _):
            base = wid * EPT + ch * CHW
            pltpu.sync_copy(src_hbm.at[pl.ds(base, CHW)], src_v)
            pltpu.sync_copy(dst_hbm.at[pl.ds(base, CHW)], dst_v)
            cp1 = pltpu.async_copy(xl_hbm.at[src_v], xlb, sem1)
            cp2 = pltpu.async_copy(xr_hbm.at[dst_v], xrb, sem2)
            cp3 = pltpu.async_copy(ee_hbm.at[pl.ds(base, CHW)], eeb, sem3)
            cp1.wait()
            cp2.wait()
            cp3.wait()

            def edge(e, _):
                s = jnp.zeros((L,), jnp.float32)
                xls = []
                for j in range(NV):
                    xlv = xlb[e, pl.ds(16 * j, 16)]
                    u = xlv + xrb[e, pl.ds(16 * j, 16)] + eeb[e, pl.ds(16 * j, 16)]
                    m = jnp.maximum(u, 0.2 * u)
                    s = s + m * attb[pl.ds(16 * j, 16)]
                    xls.append(xlv)
                ex = jnp.exp(jnp.full((L,), jnp.sum(s), jnp.float32))
                for j in range(NV):
                    xlb[e, pl.ds(16 * j, 16)] = xls[j] * ex
                exb[e, :] = ex
                return 0

            lax.fori_loop(0, CHW, edge, 0)
            pltpu.sync_copy(xlb, acc_sh.at[dst_v], add=True)
            pltpu.sync_copy(exb, den_sh.at[dst_v], add=True)
            return 0

        lax.fori_loop(0, NCH, chunk, 0)
        plsc.subcore_barrier()
        pltpu.sync_copy(acc_sh.at[pl.ds(row0, rows_per_tile)],
                        out_hbm.at[cid, pl.ds(row0, rows_per_tile)])
        pltpu.sync_copy(den_sh.at[pl.ds(row0, rows_per_tile)],
                        outd_hbm.at[cid, pl.ds(row0, rows_per_tile)])

    return sc_edge


_sc_edge_128 = _make_sc_edge(128)
_sc_edge_64 = _make_sc_edge(64)


_BN_ROWS = 1000
_BE_ROWS = 1000


def _ee4(edge_attr, we0, we1, we2, we3):
    douts = (H1, H2, H1, D_IN)

    def body(ea_ref, w0, w1, w2, w3, o0, o1, o2, o3):
        ea = ea_ref[...]
        o0[...] = jnp.dot(ea, w0[...], preferred_element_type=jnp.float32)
        o1[...] = jnp.dot(ea, w1[...], preferred_element_type=jnp.float32)
        o2[...] = jnp.dot(ea, w2[...], preferred_element_type=jnp.float32)
        o3[...] = jnp.dot(ea, w3[...], preferred_element_type=jnp.float32)

    return pl.pallas_call(
        body,
        grid=(E // _BE_ROWS,),
        in_specs=[
            pl.BlockSpec((_BE_ROWS, D_EDGE), lambda i: (i, 0)),
            pl.BlockSpec((D_EDGE, H1), lambda i: (0, 0)),
            pl.BlockSpec((D_EDGE, H2), lambda i: (0, 0)),
            pl.BlockSpec((D_EDGE, H1), lambda i: (0, 0)),
            pl.BlockSpec((D_EDGE, D_IN), lambda i: (0, 0)),
        ],
        out_specs=[pl.BlockSpec((_BE_ROWS, d), lambda i: (i, 0)) for d in douts],
        out_shape=[jax.ShapeDtypeStruct((E, d), jnp.float32) for d in douts],
    )(edge_attr, we0, we1, we2, we3)


def _mm2(x, wl, wr):
    din, dout = wl.shape

    def body(x_ref, wl_ref, wr_ref, xl_ref, xr_ref):
        xb = x_ref[...]
        xl_ref[...] = jnp.dot(xb, wl_ref[...], preferred_element_type=jnp.float32)
        xr_ref[...] = jnp.dot(xb, wr_ref[...], preferred_element_type=jnp.float32)

    return pl.pallas_call(
        body,
        grid=(N // _BN_ROWS,),
        in_specs=[
            pl.BlockSpec((_BN_ROWS, din), lambda i: (i, 0)),
            pl.BlockSpec((din, dout), lambda i: (0, 0)),
            pl.BlockSpec((din, dout), lambda i: (0, 0)),
        ],
        out_specs=[pl.BlockSpec((_BN_ROWS, dout), lambda i: (i, 0))] * 2,
        out_shape=[jax.ShapeDtypeStruct((N, dout), jnp.float32)] * 2,
    )(x, wl, wr)


def _stage(acc, den, b_prev, g, bb, wl, wr, act):
    P = b_prev.shape[1]
    dout = wl.shape[1]
    gain = float(1.0 / np.sqrt(1.0 + BN_EPS))

    def body(acc_ref, den_ref, b_ref, g_ref, bb_ref, wl_ref, wr_ref,
             h_ref, xl_ref, xr_ref):
        a = acc_ref[0] + acc_ref[1]
        d = den_ref[0, :, 0:1] + den_ref[1, :, 0:1]
        h = a / (d + 1e-16) + b_ref[...]
        if act:
            h = jnp.maximum(h, 0.0) * (g_ref[...] * gain) + bb_ref[...]
        h_ref[...] = h
        xl_ref[...] = jnp.dot(h, wl_ref[...], preferred_element_type=jnp.float32)
        xr_ref[...] = jnp.dot(h, wr_ref[...], preferred_element_type=jnp.float32)

    return pl.pallas_call(
        body,
        grid=(N // _BN_ROWS,),
        in_specs=[
            pl.BlockSpec((2, _BN_ROWS, P), lambda i: (0, i, 0)),
            pl.BlockSpec((2, _BN_ROWS, 16), lambda i: (0, i, 0)),
            pl.BlockSpec((1, P), lambda i: (0, 0)),
            pl.BlockSpec((1, P), lambda i: (0, 0)),
            pl.BlockSpec((1, P), lambda i: (0, 0)),
            pl.BlockSpec((P, dout), lambda i: (0, 0)),
            pl.BlockSpec((P, dout), lambda i: (0, 0)),
        ],
        out_specs=[
            pl.BlockSpec((_BN_ROWS, P), lambda i: (i, 0)),
            pl.BlockSpec((_BN_ROWS, dout), lambda i: (i, 0)),
            pl.BlockSpec((_BN_ROWS, dout), lambda i: (i, 0)),
        ],
        out_shape=[
            jax.ShapeDtypeStruct((N, P), jnp.float32),
            jax.ShapeDtypeStruct((N, dout), jnp.float32),
            jax.ShapeDtypeStruct((N, dout), jnp.float32),
        ],
    )(acc, den, b_prev, g, bb, wl, wr)


def _finalize(acc, den, b_prev):
    P = b_prev.shape[1]

    def body(acc_ref, den_ref, b_ref, o_ref):
        a = acc_ref[0] + acc_ref[1]
        d = den_ref[0, :, 0:1] + den_ref[1, :, 0:1]
        o_ref[...] = a / (d + 1e-16) + b_ref[...]

    return pl.pallas_call(
        body,
        grid=(N // _BN_ROWS,),
        in_specs=[
            pl.BlockSpec((2, _BN_ROWS, P), lambda i: (0, i, 0)),
            pl.BlockSpec((2, _BN_ROWS, 16), lambda i: (0, i, 0)),
            pl.BlockSpec((1, P), lambda i: (0, 0)),
        ],
        out_specs=pl.BlockSpec((_BN_ROWS, P), lambda i: (i, 0)),
        out_shape=jax.ShapeDtypeStruct((N, P), jnp.float32),
    )(acc, den, b_prev)


def kernel(x, edge_index, edge_attr, params):
    src = edge_index[0].astype(jnp.int32)
    dst = edge_index[1].astype(jnp.int32)
    p0, p1 = params['enc0'], params['enc1']
    p2, p3 = params['dec0'], params['dec1']
    row = lambda v: v.reshape(1, -1)

    ee0, ee1, ee2, ee3 = _ee4(edge_attr, p0['We'], p1['We'], p2['We'], p3['We'])
    xl0, xr0 = _mm2(x, p0['Wl'], p0['Wr'])

    acc0, den0 = _sc_edge_128(xl0, xr0, ee0, src, dst, p0['att'])
    h1, xl1, xr1 = _stage(acc0, den0, row(p0['b']),
                          row(params['bn_e0']['g']), row(params['bn_e0']['b']),
                          p1['Wl'], p1['Wr'], act=True)
    acc1, den1 = _sc_edge_64(xl1, xr1, ee1, src, dst, p1['att'])
    z, xl2, xr2 = _stage(acc1, den1, row(p1['b']),
                         row(p1['b']), row(p1['b']),
                         p2['Wl'], p2['Wr'], act=False)
    acc2, den2 = _sc_edge_128(xl2, xr2, ee2, src, dst, p2['att'])
    h2, xl3, xr3 = _stage(acc2, den2, row(p2['b']),
                          row(params['bn_d0']['g']), row(params['bn_d0']['b']),
                          p3['Wl'], p3['Wr'], act=True)
    acc3, den3 = _sc_edge_128(xl3, xr3, ee3, src, dst, p3['att'])
    x_rec = _finalize(acc3, den3, row(p3['b']))
    return z, x_rec


# speedup vs baseline: 11.5176x; 1.6386x over previous
import functools

import jax
import jax.numpy as jnp
import numpy as np
from jax import lax
from jax.experimental import pallas as pl
from jax.experimental.pallas import tpu as pltpu
from jax.experimental.pallas import tpu_sc as plsc

N = 10000
E = 320000
D_IN = 128
D_EDGE = 16
H1 = 128
H2 = 64
BN_EPS = 1e-5

NC = 2
NS = 16
L = 16
NW = NC * NS
EPT = E // NW
CHW = 40
NCH = EPT // CHW
NBUF = 2

_SC_MESH = dict(core_axis_name="c", subcore_axis_name="s")


def _make_sc_edge(P):
    NV = P // 16
    rows_per_tile = N // NS

    @functools.partial(
        pl.kernel,
        out_type=(jax.ShapeDtypeStruct((NC, N, P), jnp.float32),
                  jax.ShapeDtypeStruct((NC, N, 16), jnp.float32)),
        mesh=plsc.VectorSubcoreMesh(**_SC_MESH),
        compiler_params=pltpu.CompilerParams(use_tc_tiling_on_sc=False,
                                             needs_layout_passes=False),
        scratch_types=(
            [pltpu.VMEM((CHW,), jnp.int32)] * NBUF +
            [pltpu.VMEM((CHW,), jnp.int32)] * NBUF +
            [pltpu.VMEM((CHW, P), jnp.float32)] * NBUF +
            [pltpu.VMEM((CHW, P), jnp.float32)] * NBUF +
            [pltpu.VMEM((CHW, P), jnp.float32)] * NBUF +
            [pltpu.VMEM((CHW, 16), jnp.float32)] * NBUF +
            [pltpu.VMEM((P,), jnp.float32),
             pltpu.VMEM_SHARED((N, P), jnp.float32),
             pltpu.VMEM_SHARED((N, 16), jnp.float32)] +
            [pltpu.SemaphoreType.DMA] * (2 * NBUF)
        ),
    )
    def sc_edge(xl_hbm, xr_hbm, ee_hbm, src_hbm, dst_hbm, att_hbm,
                out_hbm, outd_hbm,
                srcv0, srcv1, dstv0, dstv1, xlb0, xlb1, xrb0, xrb1,
                eeb0, eeb1, exb0, exb1, attb, acc_sh, den_sh,
                semg0, semg1, sems0, sems1):
        cid = lax.axis_index("c")
        sid = lax.axis_index("s")
        wid = cid * NS + sid
        srcv = (srcv0, srcv1)
        dstv = (dstv0, dstv1)
        xlb = (xlb0, xlb1)
        xrb = (xrb0, xrb1)
        eeb = (eeb0, eeb1)
        exb = (exb0, exb1)
        semg = (semg0, semg1)
        sems = (sems0, sems1)

        def zrow(i, _):
            for j in range(NV):
                xlb0[i, pl.ds(16 * j, 16)] = jnp.zeros((L,), jnp.float32)
            exb0[i, :] = jnp.zeros((L,), jnp.float32)
            return 0
        lax.fori_loop(0, CHW, zrow, 0)
        row0 = sid * rows_per_tile
        nfull = rows_per_tile // CHW
        for k in range(nfull):
            pltpu.sync_copy(xlb0, acc_sh.at[pl.ds(row0 + k * CHW, CHW)])
            pltpu.sync_copy(exb0, den_sh.at[pl.ds(row0 + k * CHW, CHW)])
        rem = rows_per_tile - nfull * CHW
        if rem:
            pltpu.sync_copy(xlb0.at[pl.ds(0, rem)],
                            acc_sh.at[pl.ds(row0 + nfull * CHW, rem)])
            pltpu.sync_copy(exb0.at[pl.ds(0, rem)],
                            den_sh.at[pl.ds(row0 + nfull * CHW, rem)])
        pltpu.sync_copy(att_hbm, attb)
        plsc.subcore_barrier()
        att_vecs = tuple(attb[pl.ds(16 * j, 16)] for j in range(NV))

        def load_idx(c, b):
            base = wid * EPT + c * CHW
            pltpu.sync_copy(src_hbm.at[pl.ds(base, CHW)], srcv[b])
            pltpu.sync_copy(dst_hbm.at[pl.ds(base, CHW)], dstv[b])

        def issue_gather(c, b):
            base = wid * EPT + c * CHW
            pltpu.async_copy(xl_hbm.at[srcv[b]], xlb[b], semg[b])
            pltpu.async_copy(xr_hbm.at[dstv[b]], xrb[b], semg[b])
            pltpu.async_copy(ee_hbm.at[pl.ds(base, CHW)], eeb[b], semg[b])

        def wait_gather(c, b):
            base = wid * EPT + c * CHW
            pltpu.make_async_copy(xl_hbm.at[srcv[b]], xlb[b], semg[b]).wait()
            pltpu.make_async_copy(xr_hbm.at[dstv[b]], xrb[b], semg[b]).wait()
            pltpu.make_async_copy(ee_hbm.at[pl.ds(base, CHW)], eeb[b],
                                  semg[b]).wait()

        def issue_scatter(b):
            pltpu.async_copy(xlb[b], acc_sh.at[dstv[b]], sems[b], add=True)
            pltpu.async_copy(exb[b], den_sh.at[dstv[b]], sems[b], add=True)

        def wait_scatter(b):
            pltpu.make_async_copy(xlb[b], acc_sh.at[dstv[b]], sems[b]).wait()
            pltpu.make_async_copy(exb[b], den_sh.at[dstv[b]], sems[b]).wait()

        def compute(b):
            xlb_b, xrb_b, eeb_b, exb_b = xlb[b], xrb[b], eeb[b], exb[b]

            @plsc.parallel_loop(0, CHW, 1, unroll=2, carry=att_vecs)
            def edge(e, att):
                s = jnp.zeros((L,), jnp.float32)
                xls = []
                for j in range(NV):
                    xlv = xlb_b[e, pl.ds(16 * j, 16)]
                    u = (xlv + xrb_b[e, pl.ds(16 * j, 16)]
                         + eeb_b[e, pl.ds(16 * j, 16)])
                    m = jnp.maximum(u, 0.2 * u)
                    s = s + m * att[j]
                    xls.append(xlv)
                ex = jnp.exp(jnp.full((L,), jnp.sum(s), jnp.float32))
                for j in range(NV):
                    xlb_b[e, pl.ds(16 * j, 16)] = xls[j] * ex
                exb_b[e, :] = ex
                return att

        load_idx(0, 0)
        issue_gather(0, 0)

        @pl.loop(0, NCH, step=NBUF)
        def pipe(t):
            for b in range(NBUF):
                c = t + b
                nb = 1 - b

                @pl.when(c >= 1)
                def _():
                    wait_scatter(nb)

                @pl.when(c + 1 < NCH)
                def _():
                    load_idx(c + 1, nb)
                    issue_gather(c + 1, nb)

                wait_gather(c, b)
                compute(b)
                issue_scatter(b)

        wait_scatter((NCH - 1) % NBUF)
        plsc.subcore_barrier()
        pltpu.sync_copy(acc_sh.at[pl.ds(row0, rows_per_tile)],
                        out_hbm.at[cid, pl.ds(row0, rows_per_tile)])
        pltpu.sync_copy(den_sh.at[pl.ds(row0, rows_per_tile)],
                        outd_hbm.at[cid, pl.ds(row0, rows_per_tile)])

    return sc_edge


_sc_edge_128 = _make_sc_edge(128)
_sc_edge_64 = _make_sc_edge(64)


_BN_ROWS = 1000
_BE_ROWS = 1000


def _ee4(edge_attr, we0, we1, we2, we3):
    douts = (H1, H2, H1, D_IN)

    def body(ea_ref, w0, w1, w2, w3, o0, o1, o2, o3):
        ea = ea_ref[...]
        o0[...] = jnp.dot(ea, w0[...], preferred_element_type=jnp.float32)
        o1[...] = jnp.dot(ea, w1[...], preferred_element_type=jnp.float32)
        o2[...] = jnp.dot(ea, w2[...], preferred_element_type=jnp.float32)
        o3[...] = jnp.dot(ea, w3[...], preferred_element_type=jnp.float32)

    return pl.pallas_call(
        body,
        grid=(E // _BE_ROWS,),
        in_specs=[
            pl.BlockSpec((_BE_ROWS, D_EDGE), lambda i: (i, 0)),
            pl.BlockSpec((D_EDGE, H1), lambda i: (0, 0)),
            pl.BlockSpec((D_EDGE, H2), lambda i: (0, 0)),
            pl.BlockSpec((D_EDGE, H1), lambda i: (0, 0)),
            pl.BlockSpec((D_EDGE, D_IN), lambda i: (0, 0)),
        ],
        out_specs=[pl.BlockSpec((_BE_ROWS, d), lambda i: (i, 0)) for d in douts],
        out_shape=[jax.ShapeDtypeStruct((E, d), jnp.float32) for d in douts],
    )(edge_attr, we0, we1, we2, we3)


def _mm2(x, wl, wr):
    din, dout = wl.shape

    def body(x_ref, wl_ref, wr_ref, xl_ref, xr_ref):
        xb = x_ref[...]
        xl_ref[...] = jnp.dot(xb, wl_ref[...], preferred_element_type=jnp.float32)
        xr_ref[...] = jnp.dot(xb, wr_ref[...], preferred_element_type=jnp.float32)

    return pl.pallas_call(
        body,
        grid=(N // _BN_ROWS,),
        in_specs=[
            pl.BlockSpec((_BN_ROWS, din), lambda i: (i, 0)),
            pl.BlockSpec((din, dout), lambda i: (0, 0)),
            pl.BlockSpec((din, dout), lambda i: (0, 0)),
        ],
        out_specs=[pl.BlockSpec((_BN_ROWS, dout), lambda i: (i, 0))] * 2,
        out_shape=[jax.ShapeDtypeStruct((N, dout), jnp.float32)] * 2,
    )(x, wl, wr)


def _stage(acc, den, b_prev, g, bb, wl, wr, act):
    P = b_prev.shape[1]
    dout = wl.shape[1]
    gain = float(1.0 / np.sqrt(1.0 + BN_EPS))

    def body(acc_ref, den_ref, b_ref, g_ref, bb_ref, wl_ref, wr_ref,
             h_ref, xl_ref, xr_ref):
        a = acc_ref[0] + acc_ref[1]
        d = den_ref[0, :, 0:1] + den_ref[1, :, 0:1]
        h = a / (d + 1e-16) + b_ref[...]
        if act:
            h = jnp.maximum(h, 0.0) * (g_ref[...] * gain) + bb_ref[...]
        h_ref[...] = h
        xl_ref[...] = jnp.dot(h, wl_ref[...], preferred_element_type=jnp.float32)
        xr_ref[...] = jnp.dot(h, wr_ref[...], preferred_element_type=jnp.float32)

    return pl.pallas_call(
        body,
        grid=(N // _BN_ROWS,),
        in_specs=[
            pl.BlockSpec((2, _BN_ROWS, P), lambda i: (0, i, 0)),
            pl.BlockSpec((2, _BN_ROWS, 16), lambda i: (0, i, 0)),
            pl.BlockSpec((1, P), lambda i: (0, 0)),
            pl.BlockSpec((1, P), lambda i: (0, 0)),
            pl.BlockSpec((1, P), lambda i: (0, 0)),
            pl.BlockSpec((P, dout), lambda i: (0, 0)),
            pl.BlockSpec((P, dout), lambda i: (0, 0)),
        ],
        out_specs=[
            pl.BlockSpec((_BN_ROWS, P), lambda i: (i, 0)),
            pl.BlockSpec((_BN_ROWS, dout), lambda i: (i, 0)),
            pl.BlockSpec((_BN_ROWS, dout), lambda i: (i, 0)),
        ],
        out_shape=[
            jax.ShapeDtypeStruct((N, P), jnp.float32),
            jax.ShapeDtypeStruct((N, dout), jnp.float32),
            jax.ShapeDtypeStruct((N, dout), jnp.float32),
        ],
    )(acc, den, b_prev, g, bb, wl, wr)


def _finalize(acc, den, b_prev):
    P = b_prev.shape[1]

    def body(acc_ref, den_ref, b_ref, o_ref):
        a = acc_ref[0] + acc_ref[1]
        d = den_ref[0, :, 0:1] + den_ref[1, :, 0:1]
        o_ref[...] = a / (d + 1e-16) + b_ref[...]

    return pl.pallas_call(
        body,
        grid=(N // _BN_ROWS,),
        in_specs=[
            pl.BlockSpec((2, _BN_ROWS, P), lambda i: (0, i, 0)),
            pl.BlockSpec((2, _BN_ROWS, 16), lambda i: (0, i, 0)),
            pl.BlockSpec((1, P), lambda i: (0, 0)),
        ],
        out_specs=pl.BlockSpec((_BN_ROWS, P), lambda i: (i, 0)),
        out_shape=jax.ShapeDtypeStruct((N, P), jnp.float32),
    )(acc, den, b_prev)


def kernel(x, edge_index, edge_attr, params):
    src = edge_index[0].astype(jnp.int32)
    dst = edge_index[1].astype(jnp.int32)
    p0, p1 = params['enc0'], params['enc1']
    p2, p3 = params['dec0'], params['dec1']
    row = lambda v: v.reshape(1, -1)

    ee0, ee1, ee2, ee3 = _ee4(edge_attr, p0['We'], p1['We'], p2['We'], p3['We'])
    xl0, xr0 = _mm2(x, p0['Wl'], p0['Wr'])

    acc0, den0 = _sc_edge_128(xl0, xr0, ee0, src, dst, p0['att'])
    h1, xl1, xr1 = _stage(acc0, den0, row(p0['b']),
                          row(params['bn_e0']['g']), row(params['bn_e0']['b']),
                          p1['Wl'], p1['Wr'], act=True)
    acc1, den1 = _sc_edge_64(xl1, xr1, ee1, src, dst, p1['att'])
    z, xl2, xr2 = _stage(acc1, den1, row(p1['b']),
                         row(p1['b']), row(p1['b']),
                         p2['Wl'], p2['Wr'], act=False)
    acc2, den2 = _sc_edge_128(xl2, xr2, ee2, src, dst, p2['att'])
    h2, xl3, xr3 = _stage(acc2, den2, row(p2['b']),
                          row(params['bn_d0']['g']), row(params['bn_d0']['b']),
                          p3['Wl'], p3['Wr'], act=True)
    acc3, den3 = _sc_edge_128(xl3, xr3, ee3, src, dst, p3['att'])
    x_rec = _finalize(acc3, den3, row(p3['b']))
    return z, x_rec


# speedup vs baseline: 15.2902x; 1.3276x over previous
import functools

import jax
import jax.numpy as jnp
import numpy as np
from jax import lax
from jax.experimental import pallas as pl
from jax.experimental.pallas import tpu as pltpu
from jax.experimental.pallas import tpu_sc as plsc

N = 10000
E = 320000
D_IN = 128
D_EDGE = 16
H1 = 128
H2 = 64
BN_EPS = 1e-5

NC = 2
NS = 16
L = 16
NW = NC * NS
EPT = E // NW
CHW = 40
NCH = EPT // CHW
NBUF = 2
GBLK = 5
NBLK = NCH // GBLK

_SC_MESH = dict(core_axis_name="c", subcore_axis_name="s")


def _make_sc_edge(P):
    NV = P // 16
    rows_per_tile = N // NS

    @functools.partial(
        pl.kernel,
        out_type=(jax.ShapeDtypeStruct((NC, N, P), jnp.float32),
                  jax.ShapeDtypeStruct((NC, N, 16), jnp.float32)),
        mesh=plsc.VectorSubcoreMesh(**_SC_MESH),
        compiler_params=pltpu.CompilerParams(use_tc_tiling_on_sc=False,
                                             needs_layout_passes=False),
        scratch_types=(
            [pltpu.VMEM((GBLK, CHW), jnp.int32)] * NBUF +
            [pltpu.VMEM((GBLK, CHW), jnp.int32)] * NBUF +
            [pltpu.VMEM((CHW, P), jnp.float32)] * NBUF +
            [pltpu.VMEM((CHW, P), jnp.float32)] * NBUF +
            [pltpu.VMEM((CHW, P), jnp.float32)] * NBUF +
            [pltpu.VMEM((CHW, 16), jnp.float32)] * NBUF +
            [pltpu.VMEM((P,), jnp.float32),
             pltpu.VMEM_SHARED((N, P), jnp.float32),
             pltpu.VMEM_SHARED((N, 16), jnp.float32)] +
            [pltpu.SemaphoreType.DMA] * (2 * NBUF + 1)
        ),
    )
    def sc_edge(xl_hbm, xr_hbm, ee_hbm, src_hbm, dst_hbm, att_hbm,
                out_hbm, outd_hbm,
                srcv0, srcv1, dstv0, dstv1, xlb0, xlb1, xrb0, xrb1,
                eeb0, eeb1, exb0, exb1, attb, acc_sh, den_sh,
                semg0, semg1, sems0, sems1, semi):
        cid = lax.axis_index("c")
        sid = lax.axis_index("s")
        wid = cid * NS + sid
        srcv = (srcv0, srcv1)
        dstv = (dstv0, dstv1)
        xlb = (xlb0, xlb1)
        xrb = (xrb0, xrb1)
        eeb = (eeb0, eeb1)
        exb = (exb0, exb1)
        semg = (semg0, semg1)
        sems = (sems0, sems1)

        def zrow(i, _):
            for j in range(NV):
                xlb0[i, pl.ds(16 * j, 16)] = jnp.zeros((L,), jnp.float32)
            exb0[i, :] = jnp.zeros((L,), jnp.float32)
            return 0
        lax.fori_loop(0, CHW, zrow, 0)
        row0 = sid * rows_per_tile
        nfull = rows_per_tile // CHW
        for k in range(nfull):
            pltpu.sync_copy(xlb0, acc_sh.at[pl.ds(row0 + k * CHW, CHW)])
            pltpu.sync_copy(exb0, den_sh.at[pl.ds(row0 + k * CHW, CHW)])
        rem = rows_per_tile - nfull * CHW
        if rem:
            pltpu.sync_copy(xlb0.at[pl.ds(0, rem)],
                            acc_sh.at[pl.ds(row0 + nfull * CHW, rem)])
            pltpu.sync_copy(exb0.at[pl.ds(0, rem)],
                            den_sh.at[pl.ds(row0 + nfull * CHW, rem)])
        pltpu.sync_copy(att_hbm, attb)
        plsc.subcore_barrier()
        att_vecs = tuple(attb[pl.ds(16 * j, 16)] for j in range(NV))

        r0_tile = wid * (EPT // CHW)

        def issue_idx(kk, ib):
            r = r0_tile + kk * GBLK
            pltpu.async_copy(src_hbm.at[pl.ds(r, GBLK)], srcv[ib], semi)
            pltpu.async_copy(dst_hbm.at[pl.ds(r, GBLK)], dstv[ib], semi)

        def wait_idx(kk, ib):
            r = r0_tile + kk * GBLK
            pltpu.make_async_copy(src_hbm.at[pl.ds(r, GBLK)], srcv[ib],
                                  semi).wait()
            pltpu.make_async_copy(dst_hbm.at[pl.ds(r, GBLK)], dstv[ib],
                                  semi).wait()

        def issue_gather(c, b, ib, i):
            base = wid * EPT + c * CHW
            pltpu.async_copy(xl_hbm.at[srcv[ib].at[i]], xlb[b], semg[b])
            pltpu.async_copy(xr_hbm.at[dstv[ib].at[i]], xrb[b], semg[b])
            pltpu.async_copy(ee_hbm.at[pl.ds(base, CHW)], eeb[b], semg[b])

        def wait_gather(c, b, ib, i):
            base = wid * EPT + c * CHW
            pltpu.make_async_copy(xl_hbm.at[srcv[ib].at[i]], xlb[b],
                                  semg[b]).wait()
            pltpu.make_async_copy(xr_hbm.at[dstv[ib].at[i]], xrb[b],
                                  semg[b]).wait()
            pltpu.make_async_copy(ee_hbm.at[pl.ds(base, CHW)], eeb[b],
                                  semg[b]).wait()

        def issue_scatter(b, ib, i):
            pltpu.async_copy(xlb[b], acc_sh.at[dstv[ib].at[i]], sems[b],
                             add=True)
            pltpu.async_copy(exb[b], den_sh.at[dstv[ib].at[i]], sems[b],
                             add=True)

        def wait_scatter(b, ib, i):
            pltpu.make_async_copy(xlb[b], acc_sh.at[dstv[ib].at[i]],
                                  sems[b]).wait()
            pltpu.make_async_copy(exb[b], den_sh.at[dstv[ib].at[i]],
                                  sems[b]).wait()

        def compute(b):
            xlb_b, xrb_b, eeb_b, exb_b = xlb[b], xrb[b], eeb[b], exb[b]

            @plsc.parallel_loop(0, CHW, 1, unroll=2, carry=att_vecs)
            def edge(e, att):
                s = jnp.zeros((L,), jnp.float32)
                xls = []
                for j in range(NV):
                    xlv = xlb_b[e, pl.ds(16 * j, 16)]
                    u = (xlv + xrb_b[e, pl.ds(16 * j, 16)]
                         + eeb_b[e, pl.ds(16 * j, 16)])
                    m = jnp.maximum(u, 0.2 * u)
                    s = s + m * att[j]
                    xls.append(xlv)
                ex = jnp.exp(jnp.full((L,), jnp.sum(s), jnp.float32))
                for j in range(NV):
                    xlb_b[e, pl.ds(16 * j, 16)] = xls[j] * ex
                exb_b[e, :] = ex
                return att

        issue_idx(0, 0)
        wait_idx(0, 0)
        issue_gather(0, 0, 0, 0)

        @pl.loop(0, NBLK, step=2)
        def blocks(t2):
            for kb in range(2):
                kk = t2 + kb
                for i in range(GBLK):
                    c = kk * GBLK + i
                    b = (kb + i) % 2
                    nb = 1 - b
                    if i == 0:
                        if kb == 0:
                            @pl.when(c >= 1)
                            def _():
                                wait_scatter(nb, 1 - kb, GBLK - 1)
                        else:
                            wait_scatter(nb, 1 - kb, GBLK - 1)

                        @pl.when(kk + 1 < NBLK)
                        def _():
                            issue_idx(kk + 1, 1 - kb)
                    else:
                        wait_scatter(nb, kb, i - 1)
                    if i == GBLK - 1:
                        @pl.when(kk + 1 < NBLK)
                        def _():
                            wait_idx(kk + 1, 1 - kb)
                            issue_gather(c + 1, nb, 1 - kb, 0)
                    else:
                        issue_gather(c + 1, nb, kb, i + 1)
                    wait_gather(c, b, kb, i)
                    compute(b)
                    issue_scatter(b, kb, i)

        wait_scatter((NBLK - 1 + GBLK - 1) % 2, (NBLK - 1) % 2, GBLK - 1)
        plsc.subcore_barrier()
        pltpu.sync_copy(acc_sh.at[pl.ds(row0, rows_per_tile)],
                        out_hbm.at[cid, pl.ds(row0, rows_per_tile)])
        pltpu.sync_copy(den_sh.at[pl.ds(row0, rows_per_tile)],
                        outd_hbm.at[cid, pl.ds(row0, rows_per_tile)])

    return sc_edge


_sc_edge_128 = _make_sc_edge(128)
_sc_edge_64 = _make_sc_edge(64)


_BN_ROWS = 1000
_BE_ROWS = 1000


def _ee4(edge_attr, we0, we1, we2, we3):
    douts = (H1, H2, H1, D_IN)

    def body(ea_ref, w0, w1, w2, w3, o0, o1, o2, o3):
        ea = ea_ref[...]
        o0[...] = jnp.dot(ea, w0[...], preferred_element_type=jnp.float32)
        o1[...] = jnp.dot(ea, w1[...], preferred_element_type=jnp.float32)
        o2[...] = jnp.dot(ea, w2[...], preferred_element_type=jnp.float32)
        o3[...] = jnp.dot(ea, w3[...], preferred_element_type=jnp.float32)

    return pl.pallas_call(
        body,
        grid=(E // _BE_ROWS,),
        in_specs=[
            pl.BlockSpec((_BE_ROWS, D_EDGE), lambda i: (i, 0)),
            pl.BlockSpec((D_EDGE, H1), lambda i: (0, 0)),
            pl.BlockSpec((D_EDGE, H2), lambda i: (0, 0)),
            pl.BlockSpec((D_EDGE, H1), lambda i: (0, 0)),
            pl.BlockSpec((D_EDGE, D_IN), lambda i: (0, 0)),
        ],
        out_specs=[pl.BlockSpec((_BE_ROWS, d), lambda i: (i, 0)) for d in douts],
        out_shape=[jax.ShapeDtypeStruct((E, d), jnp.float32) for d in douts],
    )(edge_attr, we0, we1, we2, we3)


def _mm2(x, wl, wr):
    din, dout = wl.shape

    def body(x_ref, wl_ref, wr_ref, xl_ref, xr_ref):
        xb = x_ref[...]
        xl_ref[...] = jnp.dot(xb, wl_ref[...], preferred_element_type=jnp.float32)
        xr_ref[...] = jnp.dot(xb, wr_ref[...], preferred_element_type=jnp.float32)

    return pl.pallas_call(
        body,
        grid=(N // _BN_ROWS,),
        in_specs=[
            pl.BlockSpec((_BN_ROWS, din), lambda i: (i, 0)),
            pl.BlockSpec((din, dout), lambda i: (0, 0)),
            pl.BlockSpec((din, dout), lambda i: (0, 0)),
        ],
        out_specs=[pl.BlockSpec((_BN_ROWS, dout), lambda i: (i, 0))] * 2,
        out_shape=[jax.ShapeDtypeStruct((N, dout), jnp.float32)] * 2,
    )(x, wl, wr)


def _stage(acc, den, b_prev, g, bb, wl, wr, act):
    P = b_prev.shape[1]
    dout = wl.shape[1]
    gain = float(1.0 / np.sqrt(1.0 + BN_EPS))

    def body(acc_ref, den_ref, b_ref, g_ref, bb_ref, wl_ref, wr_ref,
             h_ref, xl_ref, xr_ref):
        a = acc_ref[0] + acc_ref[1]
        d = den_ref[0, :, 0:1] + den_ref[1, :, 0:1]
        h = a / (d + 1e-16) + b_ref[...]
        if act:
            h = jnp.maximum(h, 0.0) * (g_ref[...] * gain) + bb_ref[...]
        h_ref[...] = h
        xl_ref[...] = jnp.dot(h, wl_ref[...], preferred_element_type=jnp.float32)
        xr_ref[...] = jnp.dot(h, wr_ref[...], preferred_element_type=jnp.float32)

    return pl.pallas_call(
        body,
        grid=(N // _BN_ROWS,),
        in_specs=[
            pl.BlockSpec((2, _BN_ROWS, P), lambda i: (0, i, 0)),
            pl.BlockSpec((2, _BN_ROWS, 16), lambda i: (0, i, 0)),
            pl.BlockSpec((1, P), lambda i: (0, 0)),
            pl.BlockSpec((1, P), lambda i: (0, 0)),
            pl.BlockSpec((1, P), lambda i: (0, 0)),
            pl.BlockSpec((P, dout), lambda i: (0, 0)),
            pl.BlockSpec((P, dout), lambda i: (0, 0)),
        ],
        out_specs=[
            pl.BlockSpec((_BN_ROWS, P), lambda i: (i, 0)),
            pl.BlockSpec((_BN_ROWS, dout), lambda i: (i, 0)),
            pl.BlockSpec((_BN_ROWS, dout), lambda i: (i, 0)),
        ],
        out_shape=[
            jax.ShapeDtypeStruct((N, P), jnp.float32),
            jax.ShapeDtypeStruct((N, dout), jnp.float32),
            jax.ShapeDtypeStruct((N, dout), jnp.float32),
        ],
    )(acc, den, b_prev, g, bb, wl, wr)


def _finalize(acc, den, b_prev):
    P = b_prev.shape[1]

    def body(acc_ref, den_ref, b_ref, o_ref):
        a = acc_ref[0] + acc_ref[1]
        d = den_ref[0, :, 0:1] + den_ref[1, :, 0:1]
        o_ref[...] = a / (d + 1e-16) + b_ref[...]

    return pl.pallas_call(
        body,
        grid=(N // _BN_ROWS,),
        in_specs=[
            pl.BlockSpec((2, _BN_ROWS, P), lambda i: (0, i, 0)),
            pl.BlockSpec((2, _BN_ROWS, 16), lambda i: (0, i, 0)),
            pl.BlockSpec((1, P), lambda i: (0, 0)),
        ],
        out_specs=pl.BlockSpec((_BN_ROWS, P), lambda i: (i, 0)),
        out_shape=jax.ShapeDtypeStruct((N, P), jnp.float32),
    )(acc, den, b_prev)


def kernel(x, edge_index, edge_attr, params):
    src = edge_index[0].astype(jnp.int32).reshape(E // CHW, CHW)
    dst = edge_index[1].astype(jnp.int32).reshape(E // CHW, CHW)
    p0, p1 = params['enc0'], params['enc1']
    p2, p3 = params['dec0'], params['dec1']
    row = lambda v: v.reshape(1, -1)

    ee0, ee1, ee2, ee3 = _ee4(edge_attr, p0['We'], p1['We'], p2['We'], p3['We'])
    xl0, xr0 = _mm2(x, p0['Wl'], p0['Wr'])

    acc0, den0 = _sc_edge_128(xl0, xr0, ee0, src, dst, p0['att'])
    h1, xl1, xr1 = _stage(acc0, den0, row(p0['b']),
                          row(params['bn_e0']['g']), row(params['bn_e0']['b']),
                          p1['Wl'], p1['Wr'], act=True)
    acc1, den1 = _sc_edge_64(xl1, xr1, ee1, src, dst, p1['att'])
    z, xl2, xr2 = _stage(acc1, den1, row(p1['b']),
                         row(p1['b']), row(p1['b']),
                         p2['Wl'], p2['Wr'], act=False)
    acc2, den2 = _sc_edge_128(xl2, xr2, ee2, src, dst, p2['att'])
    h2, xl3, xr3 = _stage(acc2, den2, row(p2['b']),
                          row(params['bn_d0']['g']), row(params['bn_d0']['b']),
                          p3['Wl'], p3['Wr'], act=True)
    acc3, den3 = _sc_edge_128(xl3, xr3, ee3, src, dst, p3['att'])
    x_rec = _finalize(acc3, den3, row(p3['b']))
    return z, x_rec
